# baseline (speedup 1.0000x reference)
import functools

import jax
import jax.numpy as jnp
from jax.experimental import pallas as pl
from jax.experimental.pallas import tpu as pltpu

_N = 4096
_D = 16
_CBLK = 512


def _d2_block_kernel(x_ref, c_ref, o_ref):
    x = x_ref[...]
    c = c_ref[...]
    xsq = jnp.sum(x * x, axis=1)
    csq = jnp.sum(c * c, axis=1)
    g = jax.lax.dot_general(
        x, c, (((1,), (1,)), ((), ())), preferred_element_type=jnp.float32
    )
    d2 = (xsq[:, None] + csq[None, :]) - 2.0 * g
    o_ref[...] = jnp.maximum(d2, 0.0)


@functools.partial(jax.jit, static_argnames=("ncols",))
def _d2_pallas(x, centers, ncols):
    grid = (ncols // _CBLK,)
    return pl.pallas_call(
        _d2_block_kernel,
        grid=grid,
        in_specs=[
            pl.BlockSpec((_N, _D), lambda j: (0, 0)),
            pl.BlockSpec((_CBLK, _D), lambda j: (j, 0)),
        ],
        out_specs=pl.BlockSpec((_N, _CBLK), lambda j: (0, j)),
        out_shape=jax.ShapeDtypeStruct((_N, ncols), jnp.float32),
    )(x, centers)


def _cost(x, centers, ncols):
    d2 = _d2_pallas(x, centers, ncols)
    costs = jnp.min(d2, axis=1)
    indices = jnp.argmin(d2, axis=1)
    return costs, indices


def kernel(x, centers):
    costs, idx = _cost(x, centers, 512)
    s0 = jnp.sum(costs)
    max_iters = 200
    centers1 = jnp.take(x, idx, axis=0)
    costs1, idx1 = _cost(x, centers1, _N)
    s1 = jnp.sum(costs1)
    run_min = jnp.minimum(s0, s1)

    def cond(carry):
        _, _, _, _, it, stop = carry
        return jnp.logical_and(jnp.logical_not(stop), it < max_iters)

    def body(carry):
        idx_c, run_min_c, best_c, best_i, it, _ = carry
        ncenters = jnp.take(x, idx_c, axis=0)
        ncosts, nidx = _cost(x, ncenters, _N)
        s = jnp.sum(ncosts)
        stop = s == run_min_c
        better = s < run_min_c
        best_c = jnp.where(better, ncenters, best_c)
        best_i = jnp.where(better, nidx, best_i)
        run_min_c = jnp.minimum(run_min_c, s)
        return (nidx, run_min_c, best_c, best_i, it + 1, stop)

    init = (idx1, run_min, centers1, idx1, jnp.int32(1), jnp.bool_(False))
    _, _, best_c, best_i, _, _ = jax.lax.while_loop(cond, body, init)
    return (best_c, best_i)
